# baseline (speedup 1.0000x reference)
import functools

import jax
import jax.numpy as jnp
from jax import lax
from jax.experimental import pallas as pl
from jax.experimental.pallas import tpu as pltpu
from jax.experimental.pallas import tpu_sc as plsc

B, C, L = 4, 768, 1024
OUT = 159
OUTP = 256
EPS = 1e-5

NC, NS, LANES = 2, 16, 16
NW = NC * NS
EP = B * L // NW
FC = OUT // LANES + 1
TRASH = L


def _prep_body(x_ref, theta_ref, xt_ref, gg1_ref, gg2_ref,
               ll1_ref, ll2_ref, lls_ref):
    i = pl.program_id(0)
    xi = x_ref[0]

    sq = jnp.sum(xi * xi, axis=1, keepdims=True)
    g = lax.dot_general(xi, xi, (((1,), (1,)), ((), ())),
                        preferred_element_type=jnp.float32)
    d = sq - 2.0 * g + sq.T

    col = lax.broadcasted_iota(jnp.int32, (L, L), 1)
    m1 = jnp.min(d, axis=1, keepdims=True)
    a1 = jnp.min(jnp.where(d == m1, col, L), axis=1)
    d2 = jnp.where(col == a1[:, None], jnp.inf, d)
    m2 = jnp.min(d2, axis=1, keepdims=True)
    a2 = jnp.min(jnp.where(d2 == m2, col, L), axis=1)

    e_idx = lax.iota(jnp.int32, L)
    mself = jnp.logical_and(a1 != e_idx, a2 != e_idx)

    xt_ref[0] = jnp.dot(xi, theta_ref[...], preferred_element_type=jnp.float32)

    gg1_ref[0, 0] = i * L + a1
    gg2_ref[0, 0] = i * L + a2

    ll1_ref[0, 0] = a1
    ll2_ref[0, 0] = a2
    lls_ref[0, 0] = jnp.where(mself, e_idx, TRASH)


def _agg_body(tbl_hbm, gg1_hbm, gg2_hbm, out_hbm,
              i1_v, i2_v, r1_v, r2_v, sem1, sem2):
    wid = lax.axis_index("s") * NC + lax.axis_index("c")
    ebase = wid * EP

    cpa = pltpu.async_copy(gg1_hbm.at[pl.ds(ebase, EP)], i1_v, sem1)
    cpb = pltpu.async_copy(gg2_hbm.at[pl.ds(ebase, EP)], i2_v, sem2)
    cpa.wait()
    cpb.wait()

    cp1 = pltpu.async_copy(tbl_hbm.at[i1_v], r1_v, sem1)
    cp2 = pltpu.async_copy(tbl_hbm.at[i2_v], r2_v, sem2)
    cp1.wait()
    cp2.wait()

    def _edge(j, _):
        for fc in range(FC):
            sl = pl.ds(fc * LANES, LANES)
            r1_v[j, sl] = r1_v[j, sl] + r2_v[j, sl]
        return 0
    lax.fori_loop(0, EP, _edge, 0)

    pltpu.sync_copy(r1_v, out_hbm.at[pl.ds(ebase, EP)])


def _node_body(xe_ref, xt_ref, ll1_ref, ll2_ref, lls_ref, xn_ref):
    l1 = ll1_ref[0, 0]
    l2 = ll2_ref[0, 0]
    l3 = lls_ref[0, 0]
    mf = (l3 != TRASH).astype(jnp.float32)
    w = jnp.where(l3 != TRASH, 1.0 / 3.0, 0.5)
    col = lax.broadcasted_iota(jnp.int32, (L, L), 1)
    a = ((col == l1[:, None]) | (col == l2[:, None])
         | (col == l3[:, None])).astype(jnp.float32)
    deg = jnp.sum(a, axis=0)
    s = lax.dot_general(a * w[:, None], xe_ref[0], (((0,), (0,)), ((), ())),
                        preferred_element_type=jnp.float32)
    s = s + lax.dot_general(a * (w * mf)[:, None], xt_ref[0],
                            (((0,), (0,)), ((), ())),
                            preferred_element_type=jnp.float32)
    xn_ref[0] = s / deg[:, None]


def _bn_body(z_ref, gamma_ref, beta_ref, out_ref):
    z = z_ref[...]
    mean = jnp.mean(z, axis=(0, 2), keepdims=True)
    var = jnp.mean((z - mean) ** 2, axis=(0, 2), keepdims=True)
    y = (z - mean) * lax.rsqrt(var + EPS)
    y = y * gamma_ref[...][None, :, None] + beta_ref[...][None, :, None]
    out_ref[...] = jnp.maximum(y, 0.0)


@jax.jit
def kernel(x, theta, bn_gamma, bn_beta):
    xr = x.reshape(B, L, C)
    theta_pad = jnp.pad(theta, ((0, 0), (0, OUTP - OUT)))

    i32 = jnp.int32
    f32 = jnp.float32
    idx_spec = pl.BlockSpec((1, 1, L), lambda i: (i, 0, 0))
    idx_shape = jax.ShapeDtypeStruct((B, 1, L), i32)
    xt, gg1, gg2, ll1, ll2, lls = pl.pallas_call(
        _prep_body,
        grid=(B,),
        in_specs=[
            pl.BlockSpec((1, L, C), lambda i: (i, 0, 0)),
            pl.BlockSpec((C, OUTP), lambda i: (0, 0)),
        ],
        out_specs=[
            pl.BlockSpec((1, L, OUTP), lambda i: (i, 0, 0)),
            idx_spec, idx_spec, idx_spec, idx_spec, idx_spec,
        ],
        out_shape=[
            jax.ShapeDtypeStruct((B, L, OUTP), f32),
            idx_shape, idx_shape, idx_shape, idx_shape, idx_shape,
        ],
    )(xr, theta_pad)

    agg = pl.kernel(
        _agg_body,
        out_type=jax.ShapeDtypeStruct((B * L, OUTP), f32),
        mesh=plsc.VectorSubcoreMesh(core_axis_name="c", subcore_axis_name="s"),
        scratch_types=[
            pltpu.VMEM((EP,), i32),
            pltpu.VMEM((EP,), i32),
            pltpu.VMEM((EP, OUTP), f32),
            pltpu.VMEM((EP, OUTP), f32),
            pltpu.SemaphoreType.DMA,
            pltpu.SemaphoreType.DMA,
        ],
    )
    xe = agg(xt.reshape(B * L, OUTP), gg1.reshape(B * L), gg2.reshape(B * L))

    xn = pl.pallas_call(
        _node_body,
        grid=(B,),
        in_specs=[
            pl.BlockSpec((1, L, OUTP), lambda i: (i, 0, 0)),
            pl.BlockSpec((1, L, OUTP), lambda i: (i, 0, 0)),
            idx_spec, idx_spec, idx_spec,
        ],
        out_specs=pl.BlockSpec((1, L, OUTP), lambda i: (i, 0, 0)),
        out_shape=jax.ShapeDtypeStruct((B, L, OUTP), f32),
    )(xe.reshape(B, L, OUTP), xt, ll1, ll2, lls)

    z = xn.reshape(B * L, OUTP)[:, :OUT].reshape(B, OUT, L)
    y = pl.pallas_call(
        _bn_body,
        out_shape=jax.ShapeDtypeStruct((B, OUT, L), f32),
    )(z, bn_gamma, bn_beta)
    return y[..., None]


# speedup vs baseline: 1.1018x; 1.1018x over previous
import functools

import jax
import jax.numpy as jnp
import numpy as np
from jax import lax
from jax.experimental import pallas as pl
from jax.experimental.pallas import tpu as pltpu
from jax.experimental.pallas import tpu_sc as plsc

B, C, L = 4, 768, 1024
OUT = 159
OUTP = 256
EPS = 1e-5

NC, NS, LANES = 2, 16, 16
NW = NC * NS
EP = B * L // NW
FC = OUT // LANES + 1
TRASH = L

_CB = np.arange(OUT + 1, dtype=np.int64) * L
_VB = (_CB // OUT).astype(np.int32)
_FB = (_CB % OUT).astype(np.int32)
_V = np.arange(L, dtype=np.int64)
_C0 = (OUT * _V // L).astype(np.int32)
_SPLIT = (L * (_C0.astype(np.int64) + 1) - OUT * _V).astype(np.int32)
_C1 = np.minimum(_C0 + 1, OUT - 1).astype(np.int32)


def _prep_body(x_ref, theta_ref, tbl_ref, slf_ref, gg1_ref, gg2_ref,
               ll1_ref, ll2_ref, lls_ref):
    i = pl.program_id(0)
    xi = x_ref[0]

    sq = jnp.sum(xi * xi, axis=1, keepdims=True)
    g = lax.dot_general(xi, xi, (((1,), (1,)), ((), ())),
                        preferred_element_type=jnp.float32)
    d = sq - 2.0 * g + sq.T

    col = lax.broadcasted_iota(jnp.int32, (L, L), 1)
    m1 = jnp.min(d, axis=1, keepdims=True)
    a1 = jnp.min(jnp.where(d == m1, col, L), axis=1)
    d2 = jnp.where(col == a1[:, None], jnp.inf, d)
    m2 = jnp.min(d2, axis=1, keepdims=True)
    a2 = jnp.min(jnp.where(d2 == m2, col, L), axis=1)

    e_idx = lax.iota(jnp.int32, L)
    mself = jnp.logical_and(a1 != e_idx, a2 != e_idx)
    mi = mself.astype(jnp.int32)

    xt = jnp.dot(xi, theta_ref[...], preferred_element_type=jnp.float32)
    xt3 = xt * (1.0 / 3.0)
    tbl_ref[0, 0] = xt * 0.5
    tbl_ref[0, 1] = xt3
    slf_ref[0] = jnp.where(mself[:, None], xt3, 0.0)

    half = (i * 2 + mi) * L
    gg1_ref[0, 0] = half + a1
    gg2_ref[0, 0] = half + a2

    ll1_ref[0, 0] = a1
    ll2_ref[0, 0] = a2
    lls_ref[0, 0] = jnp.where(mself, e_idx, TRASH)


def _agg_body(tbl_hbm, slf_hbm, gg1_hbm, gg2_hbm, out_hbm,
              i1_v, i2_v, r1_v, r2_v, r3_v, sem1, sem2, sem3):
    wid = lax.axis_index("s") * NC + lax.axis_index("c")
    ebase = wid * EP

    cpa = pltpu.async_copy(gg1_hbm.at[pl.ds(ebase, EP)], i1_v, sem1)
    cpb = pltpu.async_copy(gg2_hbm.at[pl.ds(ebase, EP)], i2_v, sem2)
    cp3 = pltpu.async_copy(slf_hbm.at[pl.ds(ebase, EP)], r3_v, sem3)
    cpa.wait()
    cpb.wait()

    cp1 = pltpu.async_copy(tbl_hbm.at[i1_v], r1_v, sem1)
    cp2 = pltpu.async_copy(tbl_hbm.at[i2_v], r2_v, sem2)
    cp1.wait()
    cp2.wait()
    cp3.wait()

    def _edge(j, _):
        for fc in range(FC):
            sl = pl.ds(fc * LANES, LANES)
            r1_v[j, sl] = r1_v[j, sl] + r2_v[j, sl] + r3_v[j, sl]
        return 0
    lax.fori_loop(0, EP, _edge, 0)

    pltpu.sync_copy(r1_v, out_hbm.at[pl.ds(ebase, EP)])


def _finale_body(xe_ref, ll1_ref, ll2_ref, lls_ref, gamma_ref, beta_ref,
                 vb_ref, fb_ref, c0_ref, c1_ref, split_ref, out_ref):
    col = lax.broadcasted_iota(jnp.int32, (L, L), 1)
    colb = lax.broadcasted_iota(jnp.int32, (OUT + 1, L), 1)
    colf = lax.broadcasted_iota(jnp.int32, (OUT + 1, OUT), 1)
    oh_v = (colb == vb_ref[...][:, None]).astype(jnp.float32)
    oh_tri = (colb < vb_ref[...][:, None]).astype(jnp.float32)
    m_f = (colf < fb_ref[...][:, None]).astype(jnp.float32)

    xns = []
    chs = jnp.zeros((OUT,), jnp.float32)
    chq = jnp.zeros((OUT,), jnp.float32)
    for b in range(B):
        l1 = ll1_ref[b, 0]
        l2 = ll2_ref[b, 0]
        l3 = lls_ref[b, 0]
        a = ((col == l1[:, None]) | (col == l2[:, None])
             | (col == l3[:, None])).astype(jnp.float32)
        deg = jnp.sum(a, axis=0)
        s = lax.dot_general(a, xe_ref[b], (((0,), (0,)), ((), ())),
                            preferred_element_type=jnp.float32)
        xn = s[:, :OUT] / deg[:, None]
        xns.append(xn)

        xsq = xn * xn
        rows = jnp.sum(xn, axis=1, keepdims=True)
        rows2 = jnp.sum(xsq, axis=1, keepdims=True)
        p = (jnp.dot(oh_tri, rows, preferred_element_type=jnp.float32)[:, 0]
             + jnp.sum(jnp.dot(oh_v, xn,
                               preferred_element_type=jnp.float32) * m_f,
                       axis=1))
        p2 = (jnp.dot(oh_tri, rows2, preferred_element_type=jnp.float32)[:, 0]
              + jnp.sum(jnp.dot(oh_v, xsq,
                                preferred_element_type=jnp.float32) * m_f,
                        axis=1))
        chs = chs + (p[1:] - p[:-1])
        chq = chq + (p2[1:] - p2[:-1])

    n = float(B * L)
    mean = chs / n
    var = jnp.maximum(chq / n - mean * mean, 0.0)
    u = gamma_ref[...] * lax.rsqrt(var + EPS)
    w = beta_ref[...] - mean * u

    colc = lax.broadcasted_iota(jnp.int32, (L, OUT), 1)
    oh_c0 = (colc == c0_ref[...][:, None]).astype(jnp.float32)
    oh_c1 = (colc == c1_ref[...][:, None]).astype(jnp.float32)
    sel = colc < split_ref[...][:, None]
    u2 = jnp.stack([u, w], axis=1)
    m0 = jnp.dot(oh_c0, u2, preferred_element_type=jnp.float32)
    m1 = jnp.dot(oh_c1, u2, preferred_element_type=jnp.float32)
    umap = jnp.where(sel, m0[:, 0][:, None], m1[:, 0][:, None])
    wmap = jnp.where(sel, m0[:, 1][:, None], m1[:, 1][:, None])

    for b in range(B):
        out_ref[b] = jnp.maximum(xns[b] * umap + wmap, 0.0)


@jax.jit
def kernel(x, theta, bn_gamma, bn_beta):
    xr = x.reshape(B, L, C)
    theta_pad = jnp.pad(theta, ((0, 0), (0, OUTP - OUT)))

    i32 = jnp.int32
    f32 = jnp.float32
    idx_spec = pl.BlockSpec((1, 1, L), lambda i: (i, 0, 0))
    idx_shape = jax.ShapeDtypeStruct((B, 1, L), i32)
    tbl, slf, gg1, gg2, ll1, ll2, lls = pl.pallas_call(
        _prep_body,
        grid=(B,),
        in_specs=[
            pl.BlockSpec((1, L, C), lambda i: (i, 0, 0)),
            pl.BlockSpec((C, OUTP), lambda i: (0, 0)),
        ],
        out_specs=[
            pl.BlockSpec((1, 2, L, OUTP), lambda i: (i, 0, 0, 0)),
            pl.BlockSpec((1, L, OUTP), lambda i: (i, 0, 0)),
            idx_spec, idx_spec, idx_spec, idx_spec, idx_spec,
        ],
        out_shape=[
            jax.ShapeDtypeStruct((B, 2, L, OUTP), f32),
            jax.ShapeDtypeStruct((B, L, OUTP), f32),
            idx_shape, idx_shape, idx_shape, idx_shape, idx_shape,
        ],
    )(xr, theta_pad)

    agg = pl.kernel(
        _agg_body,
        out_type=jax.ShapeDtypeStruct((B * L, OUTP), f32),
        mesh=plsc.VectorSubcoreMesh(core_axis_name="c", subcore_axis_name="s"),
        scratch_types=[
            pltpu.VMEM((EP,), i32),
            pltpu.VMEM((EP,), i32),
            pltpu.VMEM((EP, OUTP), f32),
            pltpu.VMEM((EP, OUTP), f32),
            pltpu.VMEM((EP, OUTP), f32),
            pltpu.SemaphoreType.DMA,
            pltpu.SemaphoreType.DMA,
            pltpu.SemaphoreType.DMA,
        ],
    )
    xe = agg(tbl.reshape(B * 2 * L, OUTP), slf.reshape(B * L, OUTP),
             gg1.reshape(B * L), gg2.reshape(B * L))

    yo = pl.pallas_call(
        _finale_body,
        out_shape=jax.ShapeDtypeStruct((B, L, OUT), f32),
    )(xe.reshape(B, L, OUTP), ll1, ll2, lls, bn_gamma, bn_beta,
      jnp.asarray(_VB), jnp.asarray(_FB), jnp.asarray(_C0),
      jnp.asarray(_C1), jnp.asarray(_SPLIT))
    return yo.reshape(B, OUT, L, 1)


# speedup vs baseline: 1.1269x; 1.0228x over previous
import functools

import jax
import jax.numpy as jnp
import numpy as np
from jax import lax
from jax.experimental import pallas as pl
from jax.experimental.pallas import tpu as pltpu
from jax.experimental.pallas import tpu_sc as plsc

B, C, L = 4, 768, 1024
OUT = 159
OUTP = 256
EPS = 1e-5

HB = 2
NC, NS, LANES = 2, 16, 16
NW = NC * NS
EP = HB * L // NW
FC = OUT // LANES + 1
TRASH = L

_CB = np.arange(OUT + 1, dtype=np.int64) * L
_VB = (_CB // OUT).astype(np.int32)
_FB = (_CB % OUT).astype(np.int32)
_V = np.arange(L, dtype=np.int64)
_C0 = (OUT * _V // L).astype(np.int32)
_SPLIT = (L * (_C0.astype(np.int64) + 1) - OUT * _V).astype(np.int32)
_C1 = np.minimum(_C0 + 1, OUT - 1).astype(np.int32)


def _prep_body(x_ref, theta_ref, tbl_ref, slf_ref, gg1_ref, gg2_ref,
               ll1_ref, ll2_ref, lls_ref):
    i = pl.program_id(0)
    xi = x_ref[0]

    sq = jnp.sum(xi * xi, axis=1, keepdims=True)
    g = lax.dot_general(xi, xi, (((1,), (1,)), ((), ())),
                        preferred_element_type=jnp.float32)
    d = sq - 2.0 * g + sq.T

    col = lax.broadcasted_iota(jnp.int32, (L, L), 1)
    m1 = jnp.min(d, axis=1, keepdims=True)
    a1 = jnp.min(jnp.where(d == m1, col, L), axis=1)
    d2 = jnp.where(col == a1[:, None], jnp.inf, d)
    m2 = jnp.min(d2, axis=1, keepdims=True)
    a2 = jnp.min(jnp.where(d2 == m2, col, L), axis=1)

    e_idx = lax.iota(jnp.int32, L)
    mself = jnp.logical_and(a1 != e_idx, a2 != e_idx)
    mi = mself.astype(jnp.int32)

    xt = jnp.dot(xi, theta_ref[...], preferred_element_type=jnp.float32)
    xt3 = xt * (1.0 / 3.0)
    tbl_ref[0, 0] = xt * 0.5
    tbl_ref[0, 1] = xt3
    slf_ref[0] = jnp.where(mself[:, None], xt3, 0.0)

    half = (i * 2 + mi) * L
    gg1_ref[0, 0] = half + a1
    gg2_ref[0, 0] = half + a2

    ll1_ref[0, 0] = a1
    ll2_ref[0, 0] = a2
    lls_ref[0, 0] = jnp.where(mself, e_idx, TRASH)


def _agg_body(tbl_hbm, slf_hbm, gg1_hbm, gg2_hbm, out_hbm,
              i1_v, i2_v, r1_v, r2_v, r3_v, sem1, sem2, sem3):
    wid = lax.axis_index("s") * NC + lax.axis_index("c")
    ebase = wid * EP

    cpa = pltpu.async_copy(gg1_hbm.at[pl.ds(ebase, EP)], i1_v, sem1)
    cpb = pltpu.async_copy(gg2_hbm.at[pl.ds(ebase, EP)], i2_v, sem2)
    cp3 = pltpu.async_copy(slf_hbm.at[pl.ds(ebase, EP)], r3_v, sem3)
    cpa.wait()
    cpb.wait()

    cp1 = pltpu.async_copy(tbl_hbm.at[i1_v], r1_v, sem1)
    cp2 = pltpu.async_copy(tbl_hbm.at[i2_v], r2_v, sem2)
    cp1.wait()
    cp2.wait()
    cp3.wait()

    def _edge(j, _):
        for fc in range(FC):
            sl = pl.ds(fc * LANES, LANES)
            r1_v[j, sl] = r1_v[j, sl] + r2_v[j, sl] + r3_v[j, sl]
        return 0
    lax.fori_loop(0, EP, _edge, 0)

    pltpu.sync_copy(r1_v, out_hbm.at[pl.ds(ebase, EP)])


def _finale_body(xea_ref, xeb_ref, l1a_ref, l2a_ref, lsa_ref,
                 l1b_ref, l2b_ref, lsb_ref, gamma_ref, beta_ref,
                 vb_ref, fb_ref, c0_ref, c1_ref, split_ref, out_ref):
    col = lax.broadcasted_iota(jnp.int32, (L, L), 1)
    colb = lax.broadcasted_iota(jnp.int32, (OUT + 1, L), 1)
    colf = lax.broadcasted_iota(jnp.int32, (OUT + 1, OUT), 1)
    oh_v = (colb == vb_ref[...][:, None]).astype(jnp.float32)
    oh_tri = (colb < vb_ref[...][:, None]).astype(jnp.float32)
    m_f = (colf < fb_ref[...][:, None]).astype(jnp.float32)

    xns = []
    chs = jnp.zeros((OUT,), jnp.float32)
    chq = jnp.zeros((OUT,), jnp.float32)
    for b in range(B):
        xe_ref = xea_ref if b < HB else xeb_ref
        refs = (l1a_ref, l2a_ref, lsa_ref) if b < HB else \
               (l1b_ref, l2b_ref, lsb_ref)
        j = b % HB
        l1 = refs[0][j, 0]
        l2 = refs[1][j, 0]
        l3 = refs[2][j, 0]
        a = ((col == l1[:, None]) | (col == l2[:, None])
             | (col == l3[:, None])).astype(jnp.float32)
        deg = jnp.sum(a, axis=0)
        s = lax.dot_general(a, xe_ref[j], (((0,), (0,)), ((), ())),
                            preferred_element_type=jnp.float32)
        xn = s[:, :OUT] / deg[:, None]
        xns.append(xn)

        xsq = xn * xn
        rows = jnp.sum(xn, axis=1, keepdims=True)
        rows2 = jnp.sum(xsq, axis=1, keepdims=True)
        p = (jnp.dot(oh_tri, rows, preferred_element_type=jnp.float32)[:, 0]
             + jnp.sum(jnp.dot(oh_v, xn,
                               preferred_element_type=jnp.float32) * m_f,
                       axis=1))
        p2 = (jnp.dot(oh_tri, rows2, preferred_element_type=jnp.float32)[:, 0]
              + jnp.sum(jnp.dot(oh_v, xsq,
                                preferred_element_type=jnp.float32) * m_f,
                        axis=1))
        chs = chs + (p[1:] - p[:-1])
        chq = chq + (p2[1:] - p2[:-1])

    n = float(B * L)
    mean = chs / n
    var = jnp.maximum(chq / n - mean * mean, 0.0)
    u = gamma_ref[...] * lax.rsqrt(var + EPS)
    w = beta_ref[...] - mean * u

    colc = lax.broadcasted_iota(jnp.int32, (L, OUT), 1)
    oh_c0 = (colc == c0_ref[...][:, None]).astype(jnp.float32)
    oh_c1 = (colc == c1_ref[...][:, None]).astype(jnp.float32)
    sel = colc < split_ref[...][:, None]
    u2 = jnp.stack([u, w], axis=1)
    m0 = jnp.dot(oh_c0, u2, preferred_element_type=jnp.float32)
    m1 = jnp.dot(oh_c1, u2, preferred_element_type=jnp.float32)
    umap = jnp.where(sel, m0[:, 0][:, None], m1[:, 0][:, None])
    wmap = jnp.where(sel, m0[:, 1][:, None], m1[:, 1][:, None])

    for b in range(B):
        out_ref[b] = jnp.maximum(xns[b] * umap + wmap, 0.0)


@jax.jit
def kernel(x, theta, bn_gamma, bn_beta):
    xr = x.reshape(B, L, C)
    theta_pad = jnp.pad(theta, ((0, 0), (0, OUTP - OUT)))

    i32 = jnp.int32
    f32 = jnp.float32
    idx_shape = jax.ShapeDtypeStruct((HB, 1, L), i32)

    def prep(h):
        idx_spec = pl.BlockSpec((1, 1, L), lambda i: (i, 0, 0))
        return pl.pallas_call(
            _prep_body,
            grid=(HB,),
            in_specs=[
                pl.BlockSpec((1, L, C), lambda i, h=h: (i + HB * h, 0, 0)),
                pl.BlockSpec((C, OUTP), lambda i: (0, 0)),
            ],
            out_specs=[
                pl.BlockSpec((1, 2, L, OUTP), lambda i: (i, 0, 0, 0)),
                pl.BlockSpec((1, L, OUTP), lambda i: (i, 0, 0)),
                idx_spec, idx_spec, idx_spec, idx_spec, idx_spec,
            ],
            out_shape=[
                jax.ShapeDtypeStruct((HB, 2, L, OUTP), f32),
                jax.ShapeDtypeStruct((HB, L, OUTP), f32),
                idx_shape, idx_shape, idx_shape, idx_shape, idx_shape,
            ],
        )(xr, theta_pad)

    agg = pl.kernel(
        _agg_body,
        out_type=jax.ShapeDtypeStruct((HB * L, OUTP), f32),
        mesh=plsc.VectorSubcoreMesh(core_axis_name="c", subcore_axis_name="s"),
        scratch_types=[
            pltpu.VMEM((EP,), i32),
            pltpu.VMEM((EP,), i32),
            pltpu.VMEM((EP, OUTP), f32),
            pltpu.VMEM((EP, OUTP), f32),
            pltpu.VMEM((EP, OUTP), f32),
            pltpu.SemaphoreType.DMA,
            pltpu.SemaphoreType.DMA,
            pltpu.SemaphoreType.DMA,
        ],
    )

    tbl_a, slf_a, gg1a, gg2a, ll1a, ll2a, llsa = prep(0)
    xe_a = agg(tbl_a.reshape(HB * 2 * L, OUTP), slf_a.reshape(HB * L, OUTP),
               gg1a.reshape(HB * L), gg2a.reshape(HB * L))
    tbl_b, slf_b, gg1b, gg2b, ll1b, ll2b, llsb = prep(1)
    xe_b = agg(tbl_b.reshape(HB * 2 * L, OUTP), slf_b.reshape(HB * L, OUTP),
               gg1b.reshape(HB * L), gg2b.reshape(HB * L))

    yo = pl.pallas_call(
        _finale_body,
        out_shape=jax.ShapeDtypeStruct((B, L, OUT), f32),
    )(xe_a.reshape(HB, L, OUTP), xe_b.reshape(HB, L, OUTP),
      ll1a, ll2a, llsa, ll1b, ll2b, llsb, bn_gamma, bn_beta,
      jnp.asarray(_VB), jnp.asarray(_FB), jnp.asarray(_C0),
      jnp.asarray(_C1), jnp.asarray(_SPLIT))
    return yo.reshape(B, OUT, L, 1)


# speedup vs baseline: 1.1293x; 1.0021x over previous
import functools

import jax
import jax.numpy as jnp
import numpy as np
from jax import lax
from jax.experimental import pallas as pl
from jax.experimental.pallas import tpu as pltpu
from jax.experimental.pallas import tpu_sc as plsc

B, C, L = 4, 768, 1024
OUT = 159
OUTP = 256
OUTS = 160
EPS = 1e-5

HB = 2
NC, NS, LANES = 2, 16, 16
NW = NC * NS
EP = HB * L // NW
FC = OUT // LANES + 1
TRASH = L

_CB = np.arange(OUT + 1, dtype=np.int64) * L
_VB = (_CB // OUT).astype(np.int32)
_FB = (_CB % OUT).astype(np.int32)
_V = np.arange(L, dtype=np.int64)
_C0 = (OUT * _V // L).astype(np.int32)
_SPLIT = (L * (_C0.astype(np.int64) + 1) - OUT * _V).astype(np.int32)
_C1 = np.minimum(_C0 + 1, OUT - 1).astype(np.int32)


def _prep_body(x_ref, theta_ref, tbl_ref, slf_ref, gg1_ref, gg2_ref,
               ll1_ref, ll2_ref, lls_ref):
    i = pl.program_id(0)
    xi = x_ref[0]

    sq = jnp.sum(xi * xi, axis=1, keepdims=True)
    g = lax.dot_general(xi, xi, (((1,), (1,)), ((), ())),
                        preferred_element_type=jnp.float32)
    d = sq - 2.0 * g + sq.T

    col = lax.broadcasted_iota(jnp.int32, (L, L), 1)
    m1 = jnp.min(d, axis=1, keepdims=True)
    a1 = jnp.min(jnp.where(d == m1, col, L), axis=1)
    d2 = jnp.where(col == a1[:, None], jnp.inf, d)
    m2 = jnp.min(d2, axis=1, keepdims=True)
    a2 = jnp.min(jnp.where(d2 == m2, col, L), axis=1)

    e_idx = lax.iota(jnp.int32, L)
    mself = jnp.logical_and(a1 != e_idx, a2 != e_idx)
    mi = mself.astype(jnp.int32)

    xt = jnp.dot(xi, theta_ref[...], preferred_element_type=jnp.float32)
    xt3 = xt * (1.0 / 3.0)
    tbl_ref[0, 0] = xt * 0.5
    tbl_ref[0, 1] = xt3
    slf_ref[0] = jnp.where(mself[:, None], xt3[:, :OUTS], 0.0)

    half = (i * 2 + mi) * L
    gg1_ref[0, 0] = half + a1
    gg2_ref[0, 0] = half + a2

    ll1_ref[0, 0] = a1
    ll2_ref[0, 0] = a2
    lls_ref[0, 0] = jnp.where(mself, e_idx, TRASH)


def _agg_body(tbl_hbm, slf_hbm, gg1_hbm, gg2_hbm, out_hbm,
              i1_v, i2_v, r1_v, r2_v, r3_v, sem1, sem2, sem3):
    wid = lax.axis_index("s") * NC + lax.axis_index("c")
    ebase = wid * EP

    cpa = pltpu.async_copy(gg1_hbm.at[pl.ds(ebase, EP)], i1_v, sem1)
    cpb = pltpu.async_copy(gg2_hbm.at[pl.ds(ebase, EP)], i2_v, sem2)
    cp3 = pltpu.async_copy(slf_hbm.at[pl.ds(ebase, EP)], r3_v, sem3)
    cpa.wait()
    cpb.wait()

    cp1 = pltpu.async_copy(tbl_hbm.at[i1_v], r1_v, sem1)
    cp2 = pltpu.async_copy(tbl_hbm.at[i2_v], r2_v, sem2)
    cp1.wait()
    cp2.wait()
    cp3.wait()

    def _edge(j, _):
        for fc in range(FC):
            sl = pl.ds(fc * LANES, LANES)
            r3_v[j, sl] = r3_v[j, sl] + r1_v[j, sl] + r2_v[j, sl]
        return 0
    lax.fori_loop(0, EP, _edge, 0)

    pltpu.sync_copy(r3_v, out_hbm.at[pl.ds(ebase, EP)])


def _finale_body(xea_ref, xeb_ref, l1a_ref, l2a_ref, lsa_ref,
                 l1b_ref, l2b_ref, lsb_ref, gamma_ref, beta_ref,
                 vb_ref, fb_ref, c0_ref, c1_ref, split_ref, out_ref):
    col = lax.broadcasted_iota(jnp.int32, (L, L), 1)
    colb = lax.broadcasted_iota(jnp.int32, (OUT + 1, L), 1)
    colf = lax.broadcasted_iota(jnp.int32, (OUT + 1, OUT), 1)
    oh_v = (colb == vb_ref[...][:, None]).astype(jnp.float32)
    oh_tri = (colb < vb_ref[...][:, None]).astype(jnp.float32)
    m_f = (colf < fb_ref[...][:, None]).astype(jnp.float32)

    xns = []
    chs = jnp.zeros((OUT,), jnp.float32)
    chq = jnp.zeros((OUT,), jnp.float32)
    for b in range(B):
        xe_ref = xea_ref if b < HB else xeb_ref
        refs = (l1a_ref, l2a_ref, lsa_ref) if b < HB else \
               (l1b_ref, l2b_ref, lsb_ref)
        j = b % HB
        l1 = refs[0][j, 0]
        l2 = refs[1][j, 0]
        l3 = refs[2][j, 0]
        a = ((col == l1[:, None]) | (col == l2[:, None])
             | (col == l3[:, None])).astype(jnp.float32)
        deg = jnp.sum(a, axis=0)
        s = lax.dot_general(a, xe_ref[j], (((0,), (0,)), ((), ())),
                            preferred_element_type=jnp.float32)
        xn = s[:, :OUT] / deg[:, None]
        xns.append(xn)

        xsq = xn * xn
        rows = jnp.sum(xn, axis=1, keepdims=True)
        rows2 = jnp.sum(xsq, axis=1, keepdims=True)
        p = (jnp.dot(oh_tri, rows, preferred_element_type=jnp.float32)[:, 0]
             + jnp.sum(jnp.dot(oh_v, xn,
                               preferred_element_type=jnp.float32) * m_f,
                       axis=1))
        p2 = (jnp.dot(oh_tri, rows2, preferred_element_type=jnp.float32)[:, 0]
              + jnp.sum(jnp.dot(oh_v, xsq,
                                preferred_element_type=jnp.float32) * m_f,
                        axis=1))
        chs = chs + (p[1:] - p[:-1])
        chq = chq + (p2[1:] - p2[:-1])

    n = float(B * L)
    mean = chs / n
    var = jnp.maximum(chq / n - mean * mean, 0.0)
    u = gamma_ref[...] * lax.rsqrt(var + EPS)
    w = beta_ref[...] - mean * u

    colc = lax.broadcasted_iota(jnp.int32, (L, OUT), 1)
    oh_c0 = (colc == c0_ref[...][:, None]).astype(jnp.float32)
    oh_c1 = (colc == c1_ref[...][:, None]).astype(jnp.float32)
    sel = colc < split_ref[...][:, None]
    u2 = jnp.stack([u, w], axis=1)
    m0 = jnp.dot(oh_c0, u2, preferred_element_type=jnp.float32)
    m1 = jnp.dot(oh_c1, u2, preferred_element_type=jnp.float32)
    umap = jnp.where(sel, m0[:, 0][:, None], m1[:, 0][:, None])
    wmap = jnp.where(sel, m0[:, 1][:, None], m1[:, 1][:, None])

    for b in range(B):
        out_ref[b] = jnp.maximum(xns[b] * umap + wmap, 0.0)


@jax.jit
def kernel(x, theta, bn_gamma, bn_beta):
    xr = x.reshape(B, L, C)
    theta_pad = jnp.pad(theta, ((0, 0), (0, OUTP - OUT)))

    i32 = jnp.int32
    f32 = jnp.float32
    idx_shape = jax.ShapeDtypeStruct((HB, 1, L), i32)

    def prep(h):
        idx_spec = pl.BlockSpec((1, 1, L), lambda i: (i, 0, 0))
        return pl.pallas_call(
            _prep_body,
            grid=(HB,),
            in_specs=[
                pl.BlockSpec((1, L, C), lambda i, h=h: (i + HB * h, 0, 0)),
                pl.BlockSpec((C, OUTP), lambda i: (0, 0)),
            ],
            out_specs=[
                pl.BlockSpec((1, 2, L, OUTP), lambda i: (i, 0, 0, 0)),
                pl.BlockSpec((1, L, OUTS), lambda i: (i, 0, 0)),
                idx_spec, idx_spec, idx_spec, idx_spec, idx_spec,
            ],
            out_shape=[
                jax.ShapeDtypeStruct((HB, 2, L, OUTP), f32),
                jax.ShapeDtypeStruct((HB, L, OUTS), f32),
                idx_shape, idx_shape, idx_shape, idx_shape, idx_shape,
            ],
        )(xr, theta_pad)

    agg = pl.kernel(
        _agg_body,
        out_type=jax.ShapeDtypeStruct((HB * L, OUTS), f32),
        mesh=plsc.VectorSubcoreMesh(core_axis_name="c", subcore_axis_name="s"),
        scratch_types=[
            pltpu.VMEM((EP,), i32),
            pltpu.VMEM((EP,), i32),
            pltpu.VMEM((EP, OUTP), f32),
            pltpu.VMEM((EP, OUTP), f32),
            pltpu.VMEM((EP, OUTS), f32),
            pltpu.SemaphoreType.DMA,
            pltpu.SemaphoreType.DMA,
            pltpu.SemaphoreType.DMA,
        ],
    )

    tbl_a, slf_a, gg1a, gg2a, ll1a, ll2a, llsa = prep(0)
    xe_a = agg(tbl_a.reshape(HB * 2 * L, OUTP), slf_a.reshape(HB * L, OUTS),
               gg1a.reshape(HB * L), gg2a.reshape(HB * L))
    tbl_b, slf_b, gg1b, gg2b, ll1b, ll2b, llsb = prep(1)
    xe_b = agg(tbl_b.reshape(HB * 2 * L, OUTP), slf_b.reshape(HB * L, OUTS),
               gg1b.reshape(HB * L), gg2b.reshape(HB * L))

    yo = pl.pallas_call(
        _finale_body,
        out_shape=jax.ShapeDtypeStruct((B, L, OUT), f32),
    )(xe_a.reshape(HB, L, OUTS), xe_b.reshape(HB, L, OUTS),
      ll1a, ll2a, llsa, ll1b, ll2b, llsb, bn_gamma, bn_beta,
      jnp.asarray(_VB), jnp.asarray(_FB), jnp.asarray(_C0),
      jnp.asarray(_C1), jnp.asarray(_SPLIT))
    return yo.reshape(B, OUT, L, 1)
